# baseline (speedup 1.0000x reference)
import jax
import jax.numpy as jnp
from jax.experimental import pallas as pl
from jax.experimental.pallas import tpu as pltpu

B_TILE = 16
SUB = 16


def _conv_band(x3, wt_ref, b_ref, n_out):
    Bt, H, K = x3.shape
    Np = wt_ref.shape[1] // 3
    flat = x3.reshape(Bt * H, K)
    yall = jnp.dot(flat, wt_ref[...], preferred_element_type=jnp.float32)
    yall = yall.reshape(Bt, H, 3 * Np)
    y0 = yall[:, 0:H - 2, 0:n_out]
    y1 = yall[:, 1:H - 1, Np:Np + n_out]
    y2 = yall[:, 2:H, 2 * Np:2 * Np + n_out]
    return b_ref[...][None] + y0 + y1 + y2


def _res_block(x3, wt1_ref, b1_ref, wt2_ref, b2_ref):
    H = x3.shape[1]
    n1 = b1_ref.shape[1]
    n2 = b2_ref.shape[1]
    h = jnp.maximum(_conv_band(x3, wt1_ref, b1_ref, n1), 0.0)
    y = _conv_band(h, wt2_ref, b2_ref, n2)
    off = x3.shape[2] - n1
    ident = x3[:, 2:H - 2, off:off + n2].astype(jnp.float32)
    return jnp.maximum(y + ident, 0.0)


def _mb_chain_kernel(x_ref,
                     w10, b10, w20, b20,
                     w11, b11, w21, b21,
                     w12, b12, w22, b22,
                     w13, b13, w23, b23,
                     o_ref):
    for s in range(B_TILE // SUB):
        x = x_ref[s * SUB:(s + 1) * SUB]
        x = _res_block(x, w10, b10, w20, b20)
        x = _res_block(x, w11, b11, w21, b21)
        x = _res_block(x, w12, b12, w22, b22)
        x = _res_block(x, w13, b13, w23, b23)
        o_ref[s * SUB:(s + 1) * SUB] = x


def _cb_chain_kernel(x_ref, s_ref,
                     w10, b10, w20, b20,
                     w11, b11, w21, b21,
                     w12, b12, w22, b22,
                     w13, b13, w23, b23,
                     g_ref, fcb_ref, o_ref):
    for s in range(B_TILE // SUB):
        xm = x_ref[s * SUB:(s + 1) * SUB]
        sc = s_ref[s * SUB:(s + 1) * SUB]
        Bt, Hm, WC = xm.shape
        S = sc.shape[1]
        Wm = WC // (WC // Hm if False else 16)
        Wm = WC // 16
        x4 = xm.reshape(Bt, Hm, Wm, 16)
        s4 = jnp.broadcast_to(sc[:, None, None, :], (Bt, Hm, Wm, S))
        x = jnp.concatenate([x4, s4], axis=-1).reshape(Bt, Hm, Wm * (16 + S))
        x = _res_block(x, w10, b10, w20, b20)
        x = _res_block(x, w11, b11, w21, b21)
        x = _res_block(x, w12, b12, w22, b22)
        x = _res_block(x, w13, b13, w23, b23)
        t = x * g_ref[...][None]
        o_ref[s * SUB:(s + 1) * SUB] = (jnp.sum(t, axis=(1, 2))[:, None]
                                        + fcb_ref[...])


def _pack_w(wt):
    _, K, N = wt.shape
    Np = -(-N // 128) * 128
    wt = jnp.pad(wt, ((0, 0), (0, 0), (0, Np - N)))
    return jnp.swapaxes(wt, 0, 1).reshape(K, 3 * Np)


def _wspec(shape):
    nd = len(shape)
    return pl.BlockSpec(shape, lambda b: (0,) * nd)


def kernel(matrix_inputs, scalar_inputs,
           mb0_wt1, mb0_b1, mb0_wt2, mb0_b2,
           mb1_wt1, mb1_b1, mb1_wt2, mb1_b2,
           mb2_wt1, mb2_b1, mb2_wt2, mb2_b2,
           mb3_wt1, mb3_b1, mb3_wt2, mb3_b2,
           cb0_wt1, cb0_b1, cb0_wt2, cb0_b2,
           cb1_wt1, cb1_b1, cb1_wt2, cb1_b2,
           cb2_wt1, cb2_b1, cb2_wt2, cb2_b2,
           cb3_wt1, cb3_b1, cb3_wt2, cb3_b2,
           fc_w, fc_b):
    B, Cm, H, W = matrix_inputs.shape
    S = scalar_inputs.shape[1]
    Cc = Cm + S

    x = jnp.transpose(matrix_inputs, (0, 2, 3, 1)).reshape(B, H, W * Cm)
    mb = [(mb0_wt1, mb0_b1, mb0_wt2, mb0_b2),
          (mb1_wt1, mb1_b1, mb1_wt2, mb1_b2),
          (mb2_wt1, mb2_b1, mb2_wt2, mb2_b2),
          (mb3_wt1, mb3_b1, mb3_wt2, mb3_b2)]
    cb = [(cb0_wt1, cb0_b1, cb0_wt2, cb0_b2),
          (cb1_wt1, cb1_b1, cb1_wt2, cb1_b2),
          (cb2_wt1, cb2_b1, cb2_wt2, cb2_b2),
          (cb3_wt1, cb3_b1, cb3_wt2, cb3_b2)]

    grid = (B // B_TILE,)

    Hm, Wm = H - 16, W - 16
    mb_flat = [f(a) for blk in mb for f, a in zip((_pack_w, lambda v: v) * 2, blk)]
    out1 = pl.pallas_call(
        _mb_chain_kernel,
        out_shape=jax.ShapeDtypeStruct((B, Hm, Wm * Cm), jnp.float32),
        grid_spec=pltpu.PrefetchScalarGridSpec(
            num_scalar_prefetch=0,
            grid=grid,
            in_specs=[pl.BlockSpec((B_TILE, H, W * Cm), lambda b: (b, 0, 0))]
                     + [_wspec(a.shape) for a in mb_flat],
            out_specs=pl.BlockSpec((B_TILE, Hm, Wm * Cm), lambda b: (b, 0, 0)),
        ),
        compiler_params=pltpu.CompilerParams(
            dimension_semantics=("parallel",),
            vmem_limit_bytes=55 * 1024 * 1024),
    )(x, *mb_flat)


    Hf, Wf = Hm - 16, Wm - 16
    cb_flat = [f(a) for blk in cb for f, a in zip((_pack_w, lambda v: v) * 2, blk)]
    g = jnp.tile(fc_w, (1, Wf)) / float(Hf * Wf)
    out2 = pl.pallas_call(
        _cb_chain_kernel,
        out_shape=jax.ShapeDtypeStruct((B, 1), jnp.float32),
        grid_spec=pltpu.PrefetchScalarGridSpec(
            num_scalar_prefetch=0,
            grid=grid,
            in_specs=[pl.BlockSpec((B_TILE, Hm, Wm * Cm), lambda b: (b, 0, 0)),
                      pl.BlockSpec((B_TILE, S), lambda b: (b, 0))]
                     + [_wspec(a.shape) for a in cb_flat]
                     + [_wspec(g.shape), _wspec((1, 1))],
            out_specs=pl.BlockSpec((B_TILE, 1), lambda b: (b, 0)),
        ),
        compiler_params=pltpu.CompilerParams(
            dimension_semantics=("parallel",),
            vmem_limit_bytes=55 * 1024 * 1024),
    )(out1, scalar_inputs, *cb_flat, g, fc_b.reshape(1, 1))

    return out2[:, 0]


# speedup vs baseline: 1.0889x; 1.0889x over previous
import jax
import jax.numpy as jnp
from jax.experimental import pallas as pl
from jax.experimental.pallas import tpu as pltpu

B_TILE = 16
SUB = 16


def _conv_band(x3, wt_ref, b_ref):
    Bt, H, K = x3.shape
    N = wt_ref.shape[2]
    flat = x3.reshape(Bt * H, K)
    y0 = jnp.dot(flat, wt_ref[0], preferred_element_type=jnp.float32)
    y1 = jnp.dot(flat, wt_ref[1], preferred_element_type=jnp.float32)
    y2 = jnp.dot(flat, wt_ref[2], preferred_element_type=jnp.float32)
    y0 = y0.reshape(Bt, H, N)
    y1 = y1.reshape(Bt, H, N)
    y2 = y2.reshape(Bt, H, N)
    return (b_ref[...][None] + y0[:, 0:H - 2] + y1[:, 1:H - 1] + y2[:, 2:H])


def _res_block(x3, wt1_ref, b1_ref, wt2_ref, b2_ref):
    H = x3.shape[1]
    n2 = b2_ref.shape[1]
    h = jnp.maximum(_conv_band(x3, wt1_ref, b1_ref), 0.0)
    y = _conv_band(h, wt2_ref, b2_ref)
    off = x3.shape[2] - b1_ref.shape[1]
    ident = x3[:, 2:H - 2, off:off + n2].astype(jnp.float32)
    return jnp.maximum(y + ident, 0.0)


def _mb_chain_kernel(x_ref,
                     w10, b10, w20, b20,
                     w11, b11, w21, b21,
                     w12, b12, w22, b22,
                     w13, b13, w23, b23,
                     o_ref):
    for s in range(B_TILE // SUB):
        x = x_ref[s * SUB:(s + 1) * SUB]
        x = _res_block(x, w10, b10, w20, b20)
        x = _res_block(x, w11, b11, w21, b21)
        x = _res_block(x, w12, b12, w22, b22)
        x = _res_block(x, w13, b13, w23, b23)
        o_ref[s * SUB:(s + 1) * SUB] = x


def _cb_chain_kernel(x_ref, s_ref,
                     w10, b10, w20, b20,
                     w11, b11, w21, b21,
                     w12, b12, w22, b22,
                     w13, b13, w23, b23,
                     g_ref, fcb_ref, o_ref):
    for s in range(B_TILE // SUB):
        xm = x_ref[s * SUB:(s + 1) * SUB]
        sc = s_ref[s * SUB:(s + 1) * SUB]
        Bt, Hm, WC = xm.shape
        S = sc.shape[1]
        Wm = WC // (WC // Hm if False else 16)
        Wm = WC // 16
        x4 = xm.reshape(Bt, Hm, Wm, 16)
        s4 = jnp.broadcast_to(sc[:, None, None, :], (Bt, Hm, Wm, S))
        x = jnp.concatenate([x4, s4], axis=-1).reshape(Bt, Hm, Wm * (16 + S))
        x = _res_block(x, w10, b10, w20, b20)
        x = _res_block(x, w11, b11, w21, b21)
        x = _res_block(x, w12, b12, w22, b22)
        x = _res_block(x, w13, b13, w23, b23)
        t = x * g_ref[...][None]
        o_ref[s * SUB:(s + 1) * SUB] = (jnp.sum(t, axis=(1, 2))[:, None]
                                        + fcb_ref[...])


def _wspec(shape):
    nd = len(shape)
    return pl.BlockSpec(shape, lambda b: (0,) * nd)


def kernel(matrix_inputs, scalar_inputs,
           mb0_wt1, mb0_b1, mb0_wt2, mb0_b2,
           mb1_wt1, mb1_b1, mb1_wt2, mb1_b2,
           mb2_wt1, mb2_b1, mb2_wt2, mb2_b2,
           mb3_wt1, mb3_b1, mb3_wt2, mb3_b2,
           cb0_wt1, cb0_b1, cb0_wt2, cb0_b2,
           cb1_wt1, cb1_b1, cb1_wt2, cb1_b2,
           cb2_wt1, cb2_b1, cb2_wt2, cb2_b2,
           cb3_wt1, cb3_b1, cb3_wt2, cb3_b2,
           fc_w, fc_b):
    B, Cm, H, W = matrix_inputs.shape
    S = scalar_inputs.shape[1]
    Cc = Cm + S

    x = jnp.transpose(matrix_inputs, (0, 2, 3, 1)).reshape(B, H, W * Cm)
    mb = [(mb0_wt1, mb0_b1, mb0_wt2, mb0_b2),
          (mb1_wt1, mb1_b1, mb1_wt2, mb1_b2),
          (mb2_wt1, mb2_b1, mb2_wt2, mb2_b2),
          (mb3_wt1, mb3_b1, mb3_wt2, mb3_b2)]
    cb = [(cb0_wt1, cb0_b1, cb0_wt2, cb0_b2),
          (cb1_wt1, cb1_b1, cb1_wt2, cb1_b2),
          (cb2_wt1, cb2_b1, cb2_wt2, cb2_b2),
          (cb3_wt1, cb3_b1, cb3_wt2, cb3_b2)]

    grid = (B // B_TILE,)

    Hm, Wm = H - 16, W - 16
    mb_flat = [a for blk in mb for a in blk]
    out1 = pl.pallas_call(
        _mb_chain_kernel,
        out_shape=jax.ShapeDtypeStruct((B, Hm, Wm * Cm), jnp.float32),
        grid_spec=pltpu.PrefetchScalarGridSpec(
            num_scalar_prefetch=0,
            grid=grid,
            in_specs=[pl.BlockSpec((B_TILE, H, W * Cm), lambda b: (b, 0, 0))]
                     + [_wspec(a.shape) for a in mb_flat],
            out_specs=pl.BlockSpec((B_TILE, Hm, Wm * Cm), lambda b: (b, 0, 0)),
        ),
        compiler_params=pltpu.CompilerParams(
            dimension_semantics=("parallel",),
            vmem_limit_bytes=55 * 1024 * 1024),
    )(x, *mb_flat)


    Hf, Wf = Hm - 16, Wm - 16
    cb_flat = [a for blk in cb for a in blk]
    g = jnp.tile(fc_w, (1, Wf)) / float(Hf * Wf)
    out2 = pl.pallas_call(
        _cb_chain_kernel,
        out_shape=jax.ShapeDtypeStruct((B, 1), jnp.float32),
        grid_spec=pltpu.PrefetchScalarGridSpec(
            num_scalar_prefetch=0,
            grid=grid,
            in_specs=[pl.BlockSpec((B_TILE, Hm, Wm * Cm), lambda b: (b, 0, 0)),
                      pl.BlockSpec((B_TILE, S), lambda b: (b, 0))]
                     + [_wspec(a.shape) for a in cb_flat]
                     + [_wspec(g.shape), _wspec((1, 1))],
            out_specs=pl.BlockSpec((B_TILE, 1), lambda b: (b, 0)),
        ),
        compiler_params=pltpu.CompilerParams(
            dimension_semantics=("parallel",),
            vmem_limit_bytes=55 * 1024 * 1024),
    )(out1, scalar_inputs, *cb_flat, g, fc_b.reshape(1, 1))

    return out2[:, 0]


# speedup vs baseline: 1.0903x; 1.0013x over previous
import jax
import jax.numpy as jnp
from jax.experimental import pallas as pl
from jax.experimental.pallas import tpu as pltpu

B_TILE = 16


def _conv_band(x3, wt_ref, b_ref):
    Bt, H, K = x3.shape
    N = wt_ref.shape[2]
    flat = x3.reshape(Bt * H, K)
    y0 = jnp.dot(flat, wt_ref[0], preferred_element_type=jnp.float32)
    y1 = jnp.dot(flat, wt_ref[1], preferred_element_type=jnp.float32)
    y2 = jnp.dot(flat, wt_ref[2], preferred_element_type=jnp.float32)
    y0 = y0.reshape(Bt, H, N)
    y1 = y1.reshape(Bt, H, N)
    y2 = y2.reshape(Bt, H, N)
    return (b_ref[...][None] + y0[:, 0:H - 2] + y1[:, 1:H - 1] + y2[:, 2:H])


def _res_block(x3, wt1_ref, b1_ref, wt2_ref, b2_ref):
    H = x3.shape[1]
    n2 = b2_ref.shape[1]
    h = jnp.maximum(_conv_band(x3, wt1_ref, b1_ref), 0.0)
    y = _conv_band(h, wt2_ref, b2_ref)
    off = x3.shape[2] - b1_ref.shape[1]
    ident = x3[:, 2:H - 2, off:off + n2].astype(jnp.float32)
    return jnp.maximum(y + ident, 0.0)


def _mb_chain_kernel(x_ref,
                     w10, b10, w20, b20,
                     w11, b11, w21, b21,
                     w12, b12, w22, b22,
                     w13, b13, w23, b23,
                     o_ref):
    x = x_ref[...]
    x = _res_block(x, w10, b10, w20, b20)
    x = _res_block(x, w11, b11, w21, b21)
    x = _res_block(x, w12, b12, w22, b22)
    x = _res_block(x, w13, b13, w23, b23)
    o_ref[...] = x


def _cb_chain_kernel(x_ref, s_ref,
                     w10, b10, w20, b20,
                     w11, b11, w21, b21,
                     w12, b12, w22, b22,
                     w13, b13, w23, b23,
                     g_ref, fcb_ref, o_ref):
    xm = x_ref[...]
    sc = s_ref[...]
    Bt, Hm, WC = xm.shape
    S = sc.shape[1]
    Wm = Hm
    Cm = WC // Wm
    x4 = xm.reshape(Bt, Hm, Wm, Cm)
    s4 = jnp.broadcast_to(sc[:, None, None, :], (Bt, Hm, Wm, S))
    x = jnp.concatenate([x4, s4], axis=-1).reshape(Bt, Hm, Wm * (Cm + S))
    x = _res_block(x, w10, b10, w20, b20)
    x = _res_block(x, w11, b11, w21, b21)
    x = _res_block(x, w12, b12, w22, b22)
    x = _res_block(x, w13, b13, w23, b23)
    t = x * g_ref[...][None]
    o_ref[...] = jnp.sum(t, axis=(1, 2))[:, None] + fcb_ref[...]


def _wspec(shape):
    nd = len(shape)
    return pl.BlockSpec(shape, lambda b: (0,) * nd)


def kernel(matrix_inputs, scalar_inputs,
           mb0_wt1, mb0_b1, mb0_wt2, mb0_b2,
           mb1_wt1, mb1_b1, mb1_wt2, mb1_b2,
           mb2_wt1, mb2_b1, mb2_wt2, mb2_b2,
           mb3_wt1, mb3_b1, mb3_wt2, mb3_b2,
           cb0_wt1, cb0_b1, cb0_wt2, cb0_b2,
           cb1_wt1, cb1_b1, cb1_wt2, cb1_b2,
           cb2_wt1, cb2_b1, cb2_wt2, cb2_b2,
           cb3_wt1, cb3_b1, cb3_wt2, cb3_b2,
           fc_w, fc_b):
    B, Cm, H, W = matrix_inputs.shape
    S = scalar_inputs.shape[1]
    Cc = Cm + S

    x = jnp.transpose(matrix_inputs, (0, 2, 3, 1)).reshape(B, H, W * Cm)
    mb = [(mb0_wt1, mb0_b1, mb0_wt2, mb0_b2),
          (mb1_wt1, mb1_b1, mb1_wt2, mb1_b2),
          (mb2_wt1, mb2_b1, mb2_wt2, mb2_b2),
          (mb3_wt1, mb3_b1, mb3_wt2, mb3_b2)]
    cb = [(cb0_wt1, cb0_b1, cb0_wt2, cb0_b2),
          (cb1_wt1, cb1_b1, cb1_wt2, cb1_b2),
          (cb2_wt1, cb2_b1, cb2_wt2, cb2_b2),
          (cb3_wt1, cb3_b1, cb3_wt2, cb3_b2)]

    grid = (B // B_TILE,)

    Hm, Wm = H - 16, W - 16
    mb_flat = [a for blk in mb for a in blk]
    out1 = pl.pallas_call(
        _mb_chain_kernel,
        out_shape=jax.ShapeDtypeStruct((B, Hm, Wm * Cm), jnp.float32),
        grid_spec=pltpu.PrefetchScalarGridSpec(
            num_scalar_prefetch=0,
            grid=grid,
            in_specs=[pl.BlockSpec((B_TILE, H, W * Cm), lambda b: (b, 0, 0))]
                     + [_wspec(a.shape) for a in mb_flat],
            out_specs=pl.BlockSpec((B_TILE, Hm, Wm * Cm), lambda b: (b, 0, 0)),
        ),
        compiler_params=pltpu.CompilerParams(
            dimension_semantics=("parallel",),
            vmem_limit_bytes=55 * 1024 * 1024),
    )(x, *mb_flat)


    Hf, Wf = Hm - 16, Wm - 16
    cb_flat = [a for blk in cb for a in blk]
    g = jnp.tile(fc_w, (1, Wf)) / float(Hf * Wf)
    out2 = pl.pallas_call(
        _cb_chain_kernel,
        out_shape=jax.ShapeDtypeStruct((B, 1), jnp.float32),
        grid_spec=pltpu.PrefetchScalarGridSpec(
            num_scalar_prefetch=0,
            grid=grid,
            in_specs=[pl.BlockSpec((B_TILE, Hm, Wm * Cm), lambda b: (b, 0, 0)),
                      pl.BlockSpec((B_TILE, S), lambda b: (b, 0))]
                     + [_wspec(a.shape) for a in cb_flat]
                     + [_wspec(g.shape), _wspec((1, 1))],
            out_specs=pl.BlockSpec((B_TILE, 1), lambda b: (b, 0)),
        ),
        compiler_params=pltpu.CompilerParams(
            dimension_semantics=("parallel",),
            vmem_limit_bytes=55 * 1024 * 1024),
    )(out1, scalar_inputs, *cb_flat, g, fc_b.reshape(1, 1))

    return out2[:, 0]


# speedup vs baseline: 1.1012x; 1.0100x over previous
import jax
import jax.numpy as jnp
from jax.experimental import pallas as pl
from jax.experimental.pallas import tpu as pltpu

B_TILE = 16


def _conv_band(x3, wt_ref, b_ref):
    Bt, H, K = x3.shape
    N = wt_ref.shape[1]
    flat = x3.reshape(Bt * H, K)
    y0 = jnp.dot(flat, wt_ref[0:K], preferred_element_type=jnp.float32)
    y1 = jnp.dot(flat, wt_ref[K:2 * K], preferred_element_type=jnp.float32)
    y2 = jnp.dot(flat, wt_ref[2 * K:3 * K], preferred_element_type=jnp.float32)
    y0 = y0.reshape(Bt, H, N)
    y1 = y1.reshape(Bt, H, N)
    y2 = y2.reshape(Bt, H, N)
    return (b_ref[...][None] + y0[:, 0:H - 2] + y1[:, 1:H - 1] + y2[:, 2:H])


def _res_block(x3, wt1_ref, b1_ref, wt2_ref, b2_ref):
    H = x3.shape[1]
    n2 = b2_ref.shape[1]
    h = jnp.maximum(_conv_band(x3, wt1_ref, b1_ref), 0.0)
    y = _conv_band(h, wt2_ref, b2_ref)
    off = x3.shape[2] - b1_ref.shape[1]
    ident = x3[:, 2:H - 2, off:off + n2].astype(jnp.float32)
    return jnp.maximum(y + ident, 0.0)


def _mb_chain_kernel(x_ref,
                     w10, b10, w20, b20,
                     w11, b11, w21, b21,
                     w12, b12, w22, b22,
                     w13, b13, w23, b23,
                     o_ref):
    x = x_ref[...]
    x = _res_block(x, w10, b10, w20, b20)
    x = _res_block(x, w11, b11, w21, b21)
    x = _res_block(x, w12, b12, w22, b22)
    x = _res_block(x, w13, b13, w23, b23)
    o_ref[...] = x


def _cb_chain_kernel(x_ref, s_ref,
                     w10, b10, w20, b20,
                     w11, b11, w21, b21,
                     w12, b12, w22, b22,
                     w13, b13, w23, b23,
                     g_ref, fcb_ref, o_ref):
    xm = x_ref[...]
    sc = s_ref[...]
    Bt, Hm, WC = xm.shape
    S = sc.shape[1]
    Wm = Hm
    Cm = WC // Wm
    x4 = xm.reshape(Bt, Hm, Wm, Cm)
    s4 = jnp.broadcast_to(sc[:, None, None, :], (Bt, Hm, Wm, S))
    x = jnp.concatenate([x4, s4], axis=-1).reshape(Bt, Hm, Wm * (Cm + S))
    x = _res_block(x, w10, b10, w20, b20)
    x = _res_block(x, w11, b11, w21, b21)
    x = _res_block(x, w12, b12, w22, b22)
    x = _res_block(x, w13, b13, w23, b23)
    t = x * g_ref[...][None]
    o_ref[...] = jnp.sum(t, axis=(1, 2))[:, None] + fcb_ref[...]


def _wspec(shape):
    nd = len(shape)
    return pl.BlockSpec(shape, lambda b: (0,) * nd)


def kernel(matrix_inputs, scalar_inputs,
           mb0_wt1, mb0_b1, mb0_wt2, mb0_b2,
           mb1_wt1, mb1_b1, mb1_wt2, mb1_b2,
           mb2_wt1, mb2_b1, mb2_wt2, mb2_b2,
           mb3_wt1, mb3_b1, mb3_wt2, mb3_b2,
           cb0_wt1, cb0_b1, cb0_wt2, cb0_b2,
           cb1_wt1, cb1_b1, cb1_wt2, cb1_b2,
           cb2_wt1, cb2_b1, cb2_wt2, cb2_b2,
           cb3_wt1, cb3_b1, cb3_wt2, cb3_b2,
           fc_w, fc_b):
    B, Cm, H, W = matrix_inputs.shape
    S = scalar_inputs.shape[1]
    Cc = Cm + S

    x = jnp.transpose(matrix_inputs, (0, 2, 3, 1)).reshape(B, H, W * Cm)
    mb = [(mb0_wt1, mb0_b1, mb0_wt2, mb0_b2),
          (mb1_wt1, mb1_b1, mb1_wt2, mb1_b2),
          (mb2_wt1, mb2_b1, mb2_wt2, mb2_b2),
          (mb3_wt1, mb3_b1, mb3_wt2, mb3_b2)]
    cb = [(cb0_wt1, cb0_b1, cb0_wt2, cb0_b2),
          (cb1_wt1, cb1_b1, cb1_wt2, cb1_b2),
          (cb2_wt1, cb2_b1, cb2_wt2, cb2_b2),
          (cb3_wt1, cb3_b1, cb3_wt2, cb3_b2)]

    grid = (B // B_TILE,)

    Hm, Wm = H - 16, W - 16
    mb_flat = [a if a.ndim == 2 else a.reshape(-1, a.shape[2])
               for blk in mb for a in blk]
    out1 = pl.pallas_call(
        _mb_chain_kernel,
        out_shape=jax.ShapeDtypeStruct((B, Hm, Wm * Cm), jnp.float32),
        grid_spec=pltpu.PrefetchScalarGridSpec(
            num_scalar_prefetch=0,
            grid=grid,
            in_specs=[pl.BlockSpec((B_TILE, H, W * Cm), lambda b: (b, 0, 0))]
                     + [_wspec(a.shape) for a in mb_flat],
            out_specs=pl.BlockSpec((B_TILE, Hm, Wm * Cm), lambda b: (b, 0, 0)),
        ),
        compiler_params=pltpu.CompilerParams(
            dimension_semantics=("parallel",),
            vmem_limit_bytes=55 * 1024 * 1024),
    )(x, *mb_flat)


    Hf, Wf = Hm - 16, Wm - 16
    cb_flat = [a if a.ndim == 2 else a.reshape(-1, a.shape[2])
               for blk in cb for a in blk]
    g = jnp.tile(fc_w, (1, Wf)) / float(Hf * Wf)
    out2 = pl.pallas_call(
        _cb_chain_kernel,
        out_shape=jax.ShapeDtypeStruct((B, 1), jnp.float32),
        grid_spec=pltpu.PrefetchScalarGridSpec(
            num_scalar_prefetch=0,
            grid=grid,
            in_specs=[pl.BlockSpec((B_TILE, Hm, Wm * Cm), lambda b: (b, 0, 0)),
                      pl.BlockSpec((B_TILE, S), lambda b: (b, 0))]
                     + [_wspec(a.shape) for a in cb_flat]
                     + [_wspec(g.shape), _wspec((1, 1))],
            out_specs=pl.BlockSpec((B_TILE, 1), lambda b: (b, 0)),
        ),
        compiler_params=pltpu.CompilerParams(
            dimension_semantics=("parallel",),
            vmem_limit_bytes=55 * 1024 * 1024),
    )(out1, scalar_inputs, *cb_flat, g, fc_b.reshape(1, 1))

    return out2[:, 0]
